# baseline (speedup 1.0000x reference)
import functools

import jax
import jax.numpy as jnp
import numpy as np
from jax import lax
from jax.experimental import pallas as pl
from jax.experimental.pallas import tpu as pltpu
from jax.experimental.pallas import tpu_sc as plsc

_PROBABILITY = 0.1
_NUM_MODALS = 3
_B, _D = 4096, 1024
_NC, _NS = 2, 16
_NW = _NC * _NS
_RPW = _B // _NW
_CH = 32
_NCH = _RPW // _CH


def _threefry2x32(k1, k2, x0, x1):
    k1, k2 = np.uint32(k1), np.uint32(k2)
    x0, x1 = x0.astype(np.uint32).copy(), x1.astype(np.uint32).copy()
    rot = [np.array([13, 15, 26, 6], np.uint32), np.array([17, 29, 16, 24], np.uint32)]
    ks = [k1, k2, np.uint32(k1 ^ k2 ^ np.uint32(0x1BD11BDA))]
    x0, x1 = x0 + ks[0], x1 + ks[1]
    for ri, a, b, i in [(0, 1, 2, 1), (1, 2, 0, 2), (0, 0, 1, 3), (1, 1, 2, 4), (0, 2, 0, 5)]:
        for r in rot[ri]:
            x0 = (x0 + x1).astype(np.uint32)
            x1 = ((x1 << np.uint32(r)) | (x1 >> np.uint32(32 - r))).astype(np.uint32)
            x1 = (x0 ^ x1).astype(np.uint32)
        x0 = (x0 + ks[a]).astype(np.uint32)
        x1 = (x1 + ks[b] + np.uint32(i)).astype(np.uint32)
    return x0, x1


def _random_bits(k, n):
    b1, b2 = _threefry2x32(k[0], k[1], np.zeros(n, np.uint32), np.arange(n, dtype=np.uint32))
    return (b1 ^ b2).astype(np.uint32)


def _split(k):
    b1, b2 = _threefry2x32(k[0], k[1], np.zeros(2, np.uint32), np.arange(2, dtype=np.uint32))
    return (b1[0], b2[0]), (b1[1], b2[1])


def _zero_row_sets():
    k_mask, k_choice = _split((np.uint32(0), np.uint32(42)))
    bits = _random_bits(k_mask, _B)
    fb = ((bits >> np.uint32(9)) | np.uint32(0x3F800000)).astype(np.uint32)
    u = np.maximum(np.float32(0.0), fb.view(np.float32) - np.float32(1.0))
    mask = u <= np.float32(_PROBABILITY)
    k_hi, k_lo = _split(k_choice)
    hi, lo = _random_bits(k_hi, _B), _random_bits(k_lo, _B)
    span = np.uint32(_NUM_MODALS)
    mult = np.uint32(((2 ** 16) % _NUM_MODALS) ** 2 % _NUM_MODALS)
    choice = (((hi % span) * mult + lo % span) % span).astype(np.int32)
    return [mask & (choice == m) for m in range(_NUM_MODALS)]


def _build_zidx(zset):
    half = _B // _NC
    per_worker = {}
    kmax = 0
    for c in range(_NC):
        rows = [r for r in range(c * half, (c + 1) * half) if zset[r]]
        assert rows, "every SC-half has dropped rows for key 42"
        for s in range(_NS):
            lst = rows[s::_NS]
            per_worker[(c, s)] = lst if lst else [rows[0]]
            kmax = max(kmax, len(per_worker[(c, s)]))
    k = -(-kmax // 8) * 8
    tab = np.zeros((_NW, k), np.int32)
    for (c, s), lst in per_worker.items():
        lst = lst + [lst[0]] * (k - len(lst))
        tab[c * _NS + s, :] = np.array(lst, np.int32)
    return tab, k


_ZTAB2, _K = _build_zidx(_zero_row_sets()[2])
_ZT1, _K1A = _build_zidx(_zero_row_sets()[1])
_ZT2, _K2A = _build_zidx(_zero_row_sets()[2])
_KK = max(_K1A, _K2A)
_ZTAB12 = np.zeros((2 * _NW, _KK), np.int32)
_ZTAB12[:_NW, :_K1A] = _ZT1
_ZTAB12[:_NW, _K1A:] = _ZT1[:, :1]
_ZTAB12[_NW:, :_K2A] = _ZT2
_ZTAB12[_NW:, _K2A:] = _ZT2[:, :1]
_MESH = plsc.VectorSubcoreMesh(core_axis_name="c", subcore_axis_name="s")


@functools.partial(
    pl.kernel,
    mesh=_MESH,
    out_type=[jax.ShapeDtypeStruct((_B, _D), jnp.float32)],
    scratch_types=[
        pltpu.VMEM((_CH, _D), jnp.float32),
        pltpu.VMEM((_CH, _D), jnp.float32),
        pltpu.VMEM((_K, _D), jnp.float32),
        pltpu.VMEM((_K,), jnp.int32),
        pltpu.SemaphoreType.DMA,
        pltpu.SemaphoreType.DMA,
        pltpu.SemaphoreType.DMA,
        pltpu.SemaphoreType.DMA,
        pltpu.SemaphoreType.DMA,
    ],
)
def _sc_body(m2, ztab, o2, buf_a, buf_b, zbuf, idx_v,
             sem_in_a, sem_in_b, sem_out_a, sem_out_b, sem_z):
    c = lax.axis_index("c")
    s = lax.axis_index("s")
    w = c * _NS + s
    base = w * _RPW
    bufs = (buf_a, buf_b)
    sem_in, sem_out = (sem_in_a, sem_in_b), (sem_out_a, sem_out_b)
    pending_out = [None, None]
    for g in range(_NCH):
        start = base + g * _CH
        b = g % 2
        if pending_out[b] is not None:
            pending_out[b].wait()
        pltpu.async_copy(m2.at[pl.ds(start, _CH)], bufs[b], sem_in[b]).wait()
        pending_out[b] = pltpu.async_copy(
            bufs[b], o2.at[pl.ds(start, _CH)], sem_out[b])
    zeros16 = jnp.zeros((16,), jnp.float32)
    for r in range(_K):
        for cc in range(_D // 16):
            zbuf[r, pl.ds(cc * 16, 16)] = zeros16
    for b in (0, 1):
        if pending_out[b] is not None:
            pending_out[b].wait()
    plsc.subcore_barrier()
    pltpu.sync_copy(ztab.at[w], idx_v)
    pltpu.async_copy(zbuf, o2.at[idx_v], sem_z).wait()


@functools.partial(
    pl.kernel,
    mesh=_MESH,
    out_type=[jax.ShapeDtypeStruct((_B, _D), jnp.float32)] * 2,
    scratch_types=[
        pltpu.VMEM((_CH, _D), jnp.float32),
        pltpu.VMEM((_CH, _D), jnp.float32),
        pltpu.VMEM((_KK, _D), jnp.float32),
        pltpu.VMEM((_KK,), jnp.int32),
        pltpu.SemaphoreType.DMA,
        pltpu.SemaphoreType.DMA,
        pltpu.SemaphoreType.DMA,
        pltpu.SemaphoreType.DMA,
        pltpu.SemaphoreType.DMA,
    ],
)
def _sc_body12(m1, m2, ztab, o1, o2, buf_a, buf_b, zbuf, idx_v,
               sem_in_a, sem_in_b, sem_out_a, sem_out_b, sem_z):
    c = lax.axis_index("c")
    s = lax.axis_index("s")
    w = c * _NS + s
    base = w * _RPW
    ins, outs = (m1, m2), (o1, o2)
    bufs = (buf_a, buf_b)
    sem_in, sem_out = (sem_in_a, sem_in_b), (sem_out_a, sem_out_b)
    pending_out = [None, None]
    for g in range(2 * _NCH):
        m, ch = divmod(g, _NCH)
        start = base + ch * _CH
        b = g % 2
        if pending_out[b] is not None:
            pending_out[b].wait()
        pltpu.async_copy(ins[m].at[pl.ds(start, _CH)], bufs[b], sem_in[b]).wait()
        pending_out[b] = pltpu.async_copy(
            bufs[b], outs[m].at[pl.ds(start, _CH)], sem_out[b])
    zeros16 = jnp.zeros((16,), jnp.float32)
    for r in range(_KK):
        for cc in range(_D // 16):
            zbuf[r, pl.ds(cc * 16, 16)] = zeros16
    for b in (0, 1):
        if pending_out[b] is not None:
            pending_out[b].wait()
    plsc.subcore_barrier()
    for m, dst in enumerate(outs):
        pltpu.sync_copy(ztab.at[m * _NW + w], idx_v)
        pltpu.async_copy(zbuf, dst.at[idx_v], sem_z).wait()


def _tc_body1(m0, z0, o0):
    o0[...] = jnp.where(z0[...] != 0, jnp.float32(0), m0[...])


def _tc_call1(modal0):
    B, D = modal0.shape
    z0 = jnp.asarray(_zero_row_sets()[0].astype(np.float32))[:, None]
    row_spec = pl.BlockSpec((_TC_BLK, D), lambda i: (i, 0))
    msk_spec = pl.BlockSpec((_TC_BLK, 1), lambda i: (i, 0))
    return pl.pallas_call(
        _tc_body1,
        grid=(B // _TC_BLK,),
        in_specs=[row_spec, msk_spec],
        out_specs=row_spec,
        out_shape=jax.ShapeDtypeStruct((B, D), modal0.dtype),
    )(modal0, z0)


def _tc_body(m0, m1, z0, z1, o0, o1):
    o0[...] = jnp.where(z0[...] != 0, jnp.float32(0), m0[...])
    o1[...] = jnp.where(z1[...] != 0, jnp.float32(0), m1[...])


_TC_BLK = 512


def _tc_call(modal0, modal1):
    B, D = modal0.shape
    zsets = _zero_row_sets()
    z0, z1 = (jnp.asarray(z.astype(np.float32))[:, None] for z in zsets[:2])
    row_spec = pl.BlockSpec((_TC_BLK, D), lambda i: (i, 0))
    msk_spec = pl.BlockSpec((_TC_BLK, 1), lambda i: (i, 0))
    return pl.pallas_call(
        _tc_body,
        grid=(B // _TC_BLK,),
        in_specs=[row_spec, row_spec, msk_spec, msk_spec],
        out_specs=[row_spec, row_spec],
        out_shape=[jax.ShapeDtypeStruct((B, D), modal0.dtype)] * 2,
    )(modal0, modal1, z0, z1)


@jax.jit
def kernel(modal0, modal1, modal2):
    o0 = _tc_call1(modal0)
    o1, o2 = _sc_body12(modal1, modal2, jnp.asarray(_ZTAB12))
    return (o0, o1, o2)


# speedup vs baseline: 1.0866x; 1.0866x over previous
import functools

import jax
import jax.numpy as jnp
import numpy as np
from jax import lax
from jax.experimental import pallas as pl
from jax.experimental.pallas import tpu as pltpu
from jax.experimental.pallas import tpu_sc as plsc

_PROBABILITY = 0.1
_NUM_MODALS = 3
_B, _D = 4096, 1024
_NC, _NS = 2, 16
_NW = _NC * _NS
_RPW = _B // _NW
_CH = 32
_NCH = _RPW // _CH


def _threefry2x32(k1, k2, x0, x1):
    k1, k2 = np.uint32(k1), np.uint32(k2)
    x0, x1 = x0.astype(np.uint32).copy(), x1.astype(np.uint32).copy()
    rot = [np.array([13, 15, 26, 6], np.uint32), np.array([17, 29, 16, 24], np.uint32)]
    ks = [k1, k2, np.uint32(k1 ^ k2 ^ np.uint32(0x1BD11BDA))]
    x0, x1 = x0 + ks[0], x1 + ks[1]
    for ri, a, b, i in [(0, 1, 2, 1), (1, 2, 0, 2), (0, 0, 1, 3), (1, 1, 2, 4), (0, 2, 0, 5)]:
        for r in rot[ri]:
            x0 = (x0 + x1).astype(np.uint32)
            x1 = ((x1 << np.uint32(r)) | (x1 >> np.uint32(32 - r))).astype(np.uint32)
            x1 = (x0 ^ x1).astype(np.uint32)
        x0 = (x0 + ks[a]).astype(np.uint32)
        x1 = (x1 + ks[b] + np.uint32(i)).astype(np.uint32)
    return x0, x1


def _random_bits(k, n):
    b1, b2 = _threefry2x32(k[0], k[1], np.zeros(n, np.uint32), np.arange(n, dtype=np.uint32))
    return (b1 ^ b2).astype(np.uint32)


def _split(k):
    b1, b2 = _threefry2x32(k[0], k[1], np.zeros(2, np.uint32), np.arange(2, dtype=np.uint32))
    return (b1[0], b2[0]), (b1[1], b2[1])


def _zero_row_sets():
    k_mask, k_choice = _split((np.uint32(0), np.uint32(42)))
    bits = _random_bits(k_mask, _B)
    fb = ((bits >> np.uint32(9)) | np.uint32(0x3F800000)).astype(np.uint32)
    u = np.maximum(np.float32(0.0), fb.view(np.float32) - np.float32(1.0))
    mask = u <= np.float32(_PROBABILITY)
    k_hi, k_lo = _split(k_choice)
    hi, lo = _random_bits(k_hi, _B), _random_bits(k_lo, _B)
    span = np.uint32(_NUM_MODALS)
    mult = np.uint32(((2 ** 16) % _NUM_MODALS) ** 2 % _NUM_MODALS)
    choice = (((hi % span) * mult + lo % span) % span).astype(np.int32)
    return [mask & (choice == m) for m in range(_NUM_MODALS)]


def _build_zidx(zset):
    half = _B // _NC
    per_worker = {}
    kmax = 0
    for c in range(_NC):
        rows = [r for r in range(c * half, (c + 1) * half) if zset[r]]
        assert rows, "every SC-half has dropped rows for key 42"
        for s in range(_NS):
            lst = rows[s::_NS]
            per_worker[(c, s)] = lst if lst else [rows[0]]
            kmax = max(kmax, len(per_worker[(c, s)]))
    k = -(-kmax // 8) * 8
    tab = np.zeros((_NW, k), np.int32)
    for (c, s), lst in per_worker.items():
        lst = lst + [lst[0]] * (k - len(lst))
        tab[c * _NS + s, :] = np.array(lst, np.int32)
    return tab, k


_ZTAB2, _K = _build_zidx(_zero_row_sets()[2])
_ZT1, _K1A = _build_zidx(_zero_row_sets()[1])
_ZT2, _K2A = _build_zidx(_zero_row_sets()[2])
_KK = max(_K1A, _K2A)
_ZTAB12 = np.zeros((2 * _NW, _KK), np.int32)
_ZTAB12[:_NW, :_K1A] = _ZT1
_ZTAB12[:_NW, _K1A:] = _ZT1[:, :1]
_ZTAB12[_NW:, :_K2A] = _ZT2
_ZTAB12[_NW:, _K2A:] = _ZT2[:, :1]
_MESH = plsc.VectorSubcoreMesh(core_axis_name="c", subcore_axis_name="s")


@functools.partial(
    pl.kernel,
    mesh=_MESH,
    out_type=[jax.ShapeDtypeStruct((_B, _D), jnp.float32)],
    scratch_types=[
        pltpu.VMEM((_CH, _D), jnp.float32),
        pltpu.VMEM((_CH, _D), jnp.float32),
        pltpu.VMEM((_K, _D), jnp.float32),
        pltpu.VMEM((_K,), jnp.int32),
        pltpu.SemaphoreType.DMA,
        pltpu.SemaphoreType.DMA,
        pltpu.SemaphoreType.DMA,
        pltpu.SemaphoreType.DMA,
        pltpu.SemaphoreType.DMA,
    ],
)
def _sc_body(m2, ztab, o2, buf_a, buf_b, zbuf, idx_v,
             sem_in_a, sem_in_b, sem_out_a, sem_out_b, sem_z):
    c = lax.axis_index("c")
    s = lax.axis_index("s")
    w = c * _NS + s
    base = w * _RPW
    bufs = (buf_a, buf_b)
    sem_in, sem_out = (sem_in_a, sem_in_b), (sem_out_a, sem_out_b)
    pending_out = [None, None]
    for g in range(_NCH):
        start = base + g * _CH
        b = g % 2
        if pending_out[b] is not None:
            pending_out[b].wait()
        pltpu.async_copy(m2.at[pl.ds(start, _CH)], bufs[b], sem_in[b]).wait()
        pending_out[b] = pltpu.async_copy(
            bufs[b], o2.at[pl.ds(start, _CH)], sem_out[b])
    zeros16 = jnp.zeros((16,), jnp.float32)
    for r in range(_K):
        for cc in range(_D // 16):
            zbuf[r, pl.ds(cc * 16, 16)] = zeros16
    for b in (0, 1):
        if pending_out[b] is not None:
            pending_out[b].wait()
    plsc.subcore_barrier()
    pltpu.sync_copy(ztab.at[w], idx_v)
    pltpu.async_copy(zbuf, o2.at[idx_v], sem_z).wait()


@functools.partial(
    pl.kernel,
    mesh=_MESH,
    out_type=[jax.ShapeDtypeStruct((_B, _D), jnp.float32)] * 2,
    scratch_types=[
        pltpu.VMEM((_CH, _D), jnp.float32),
        pltpu.VMEM((_CH, _D), jnp.float32),
        pltpu.VMEM((_KK, _D), jnp.float32),
        pltpu.VMEM((_KK,), jnp.int32),
        pltpu.SemaphoreType.DMA,
        pltpu.SemaphoreType.DMA,
        pltpu.SemaphoreType.DMA,
        pltpu.SemaphoreType.DMA,
        pltpu.SemaphoreType.DMA,
    ],
)
def _sc_body12(m1, m2, ztab, o1, o2, buf_a, buf_b, zbuf, idx_v,
               sem_in_a, sem_in_b, sem_out_a, sem_out_b, sem_z):
    c = lax.axis_index("c")
    s = lax.axis_index("s")
    w = c * _NS + s
    base = w * _RPW
    ins, outs = (m1, m2), (o1, o2)
    bufs = (buf_a, buf_b)
    sem_in, sem_out = (sem_in_a, sem_in_b), (sem_out_a, sem_out_b)
    pending_out = [None, None]
    for g in range(2 * _NCH):
        m, ch = divmod(g, _NCH)
        start = base + ch * _CH
        b = g % 2
        if pending_out[b] is not None:
            pending_out[b].wait()
        pltpu.async_copy(ins[m].at[pl.ds(start, _CH)], bufs[b], sem_in[b]).wait()
        pending_out[b] = pltpu.async_copy(
            bufs[b], outs[m].at[pl.ds(start, _CH)], sem_out[b])
    zeros16 = jnp.zeros((16,), jnp.float32)
    for r in range(_KK):
        for cc in range(_D // 16):
            zbuf[r, pl.ds(cc * 16, 16)] = zeros16
    for b in (0, 1):
        if pending_out[b] is not None:
            pending_out[b].wait()
    plsc.subcore_barrier()
    for m, dst in enumerate(outs):
        pltpu.sync_copy(ztab.at[m * _NW + w], idx_v)
        pltpu.async_copy(zbuf, dst.at[idx_v], sem_z).wait()


def _tc_body1(m0, z0, o0):
    o0[...] = jnp.where(z0[...] != 0, jnp.float32(0), m0[...])


def _tc_call1(modal0):
    B, D = modal0.shape
    z0 = jnp.asarray(_zero_row_sets()[0].astype(np.float32))[:, None]
    row_spec = pl.BlockSpec((_TC_BLK, D), lambda i: (i, 0))
    msk_spec = pl.BlockSpec((_TC_BLK, 1), lambda i: (i, 0))
    return pl.pallas_call(
        _tc_body1,
        grid=(B // _TC_BLK,),
        in_specs=[row_spec, msk_spec],
        out_specs=row_spec,
        out_shape=jax.ShapeDtypeStruct((B, D), modal0.dtype),
    )(modal0, z0)


def _tc_body(m0, m1, z0, z1, o0, o1):
    o0[...] = jnp.where(z0[...] != 0, jnp.float32(0), m0[...])
    o1[...] = jnp.where(z1[...] != 0, jnp.float32(0), m1[...])


_TC_BLK = 1024


def _tc_call(modal0, modal1):
    B, D = modal0.shape
    zsets = _zero_row_sets()
    z0, z1 = (jnp.asarray(z.astype(np.float32))[:, None] for z in zsets[:2])
    row_spec = pl.BlockSpec((_TC_BLK, D), lambda i: (i, 0))
    msk_spec = pl.BlockSpec((_TC_BLK, 1), lambda i: (i, 0))
    return pl.pallas_call(
        _tc_body,
        grid=(B // _TC_BLK,),
        in_specs=[row_spec, row_spec, msk_spec, msk_spec],
        out_specs=[row_spec, row_spec],
        out_shape=[jax.ShapeDtypeStruct((B, D), modal0.dtype)] * 2,
    )(modal0, modal1, z0, z1)


@jax.jit
def kernel(modal0, modal1, modal2):
    o0, o1 = _tc_call(modal0, modal1)
    (o2,) = _sc_body(modal2, jnp.asarray(_ZTAB2))
    return (o0, o1, o2)
